# baseline (speedup 1.0000x reference)
import functools

import jax
import jax.numpy as jnp
from jax import lax
from jax.experimental import pallas as pl
from jax.experimental.pallas import tpu as pltpu
from jax.experimental.pallas import tpu_sc as plsc

N = 65536
D = 512
M = 8192
BR = 1024
ROWS = N // 128


def _score_body(feat_ref, coord_ref, W1_ref, b1_ref, W2_ref, b2_ref, out_ref):
    x = jnp.concatenate([feat_ref[...], coord_ref[...]], axis=1)
    h = lax.dot_general(x, W1_ref[...], (((1,), (0,)), ((), ())),
                        preferred_element_type=jnp.float32)
    h = jnp.maximum(h + b1_ref[...], 0.0)
    s = lax.dot_general(W2_ref[...], h, (((0,), (1,)), ((), ())),
                        preferred_element_type=jnp.float32)
    out_ref[...] = (s + b2_ref[...])[None]


def _scores(feat, coord, W1, b1, W2, b2):
    grid = N // BR
    out = pl.pallas_call(
        _score_body,
        grid=(grid,),
        in_specs=[pl.BlockSpec((BR, D), lambda i: (i, 0)),
                  pl.BlockSpec((BR, 3), lambda i: (i, 0)),
                  pl.BlockSpec((D + 3, D), lambda i: (0, 0)),
                  pl.BlockSpec((1, D), lambda i: (0, 0)),
                  pl.BlockSpec((D, 1), lambda i: (0, 0)),
                  pl.BlockSpec((1, 1), lambda i: (0, 0))],
        out_specs=pl.BlockSpec((1, 1, BR), lambda i: (i, 0, 0)),
        out_shape=jax.ShapeDtypeStruct((grid, 1, BR), jnp.float32),
    )(feat, coord, W1, b1.reshape(1, D), W2, b2.reshape(1, 1))
    return out.reshape(ROWS, 128)


def _cmp_ge(ka, ia, kb, ib):
    return (ka > kb) | ((ka == kb) & (ia < ib))


def _partner(x, s):
    if s < 128:
        lo = pltpu.roll(x, 128 - s, 1)
        hi = pltpu.roll(x, s, 1)
        cbit = (lax.broadcasted_iota(jnp.int32, x.shape, 1) & s) == 0
    else:
        t = s // 128
        lo = pltpu.roll(x, ROWS - t, 0)
        hi = pltpu.roll(x, t, 0)
        cbit = (lax.broadcasted_iota(jnp.int32, x.shape, 0) & t) == 0
    return jnp.where(cbit, lo, hi)


def _sort_body(score_ref, idx_out_ref):
    keys = score_ref[...]
    ridx = lax.broadcasted_iota(jnp.int32, keys.shape, 0)
    cidx = lax.broadcasted_iota(jnp.int32, keys.shape, 1)
    idx = ridx * 128 + cidx
    for kk in range(1, 17):
        k = 1 << kk
        if k < 128:
            region_desc = (cidx & k) == 0
        else:
            region_desc = (ridx & (k // 128)) == 0
        for j in range(kk - 1, -1, -1):
            s = 1 << j
            if s < 128:
                low = (cidx & s) == 0
            else:
                low = (ridx & (s // 128)) == 0
            pk = _partner(keys, s)
            pi = _partner(idx, s)
            take_max = region_desc == low
            self_first = _cmp_ge(keys, idx, pk, pi)
            keep = take_max == self_first
            keys = jnp.where(keep, keys, pk)
            idx = jnp.where(keep, idx, pi)
    idx_out_ref[...] = idx[:M // 128, :]


def _top_idx(scores):
    return pl.pallas_call(
        _sort_body,
        in_specs=[pl.BlockSpec((ROWS, 128), lambda: (0, 0))],
        out_specs=pl.BlockSpec((M // 128, 128), lambda: (0, 0)),
        out_shape=jax.ShapeDtypeStruct((M // 128, 128), jnp.int32),
    )(scores)


@functools.cache
def _make_gather():
    info = plsc.get_sparse_core_info()
    NC, NS = info.num_cores, info.num_subcores
    NW = NC * NS
    b_per_w = M // NW
    CH = 128
    NCH = b_per_w // CH
    mesh = plsc.VectorSubcoreMesh(core_axis_name="c", subcore_axis_name="s")

    @functools.partial(
        pl.kernel, mesh=mesh,
        out_type=[jax.ShapeDtypeStruct((M, D), jnp.float32),
                  jax.ShapeDtypeStruct((M, 128), jnp.float32)],
        scratch_types=[pltpu.VMEM((b_per_w,), jnp.int32),
                       pltpu.VMEM((CH, D), jnp.float32),
                       pltpu.VMEM((b_per_w, 128), jnp.float32),
                       pltpu.SemaphoreType.DMA],
    )
    def gather(feat_hbm, coordp_hbm, idx_hbm, out_f_hbm, out_c_hbm,
               idx_v, rows_v, crows_v, sem):
        wid = lax.axis_index("s") * NC + lax.axis_index("c")
        base = wid * b_per_w
        pltpu.sync_copy(idx_hbm.at[pl.ds(base, b_per_w)], idx_v)
        for ch in range(NCH):
            idx_ch = idx_v.at[pl.ds(ch * CH, CH)]
            pltpu.async_copy(feat_hbm.at[idx_ch], rows_v, sem).wait()
            pltpu.sync_copy(rows_v, out_f_hbm.at[pl.ds(base + ch * CH, CH)])
            pltpu.async_copy(coordp_hbm.at[idx_ch],
                             crows_v.at[pl.ds(ch * CH, CH)], sem).wait()
        pltpu.sync_copy(crows_v, out_c_hbm.at[pl.ds(base, b_per_w)])

    return gather


def kernel(feat_list, coord_list, W1, b1, W2, b2):
    feat = feat_list[0]
    coord = coord_list[0]
    scores = _scores(feat, coord, W1, b1, W2, b2)
    top = _top_idx(scores).reshape(M)
    coordp = jnp.pad(coord, ((0, 0), (0, 125)))
    out_f, out_c = _make_gather()(feat, coordp, top)
    return out_f[None], out_c[:, :3][None]


# speedup vs baseline: 1.1251x; 1.1251x over previous
import functools

import jax
import jax.numpy as jnp
from jax import lax
from jax.experimental import pallas as pl
from jax.experimental.pallas import tpu as pltpu
from jax.experimental.pallas import tpu_sc as plsc

N = 65536
D = 512
M = 8192
BR = 1024
ROWS = N // 128


def _score_body(feat_ref, coord_ref, W1_ref, b1_ref, W2_ref, b2_ref, out_ref):
    x = jnp.concatenate([feat_ref[...], coord_ref[...]], axis=1)
    h = lax.dot_general(x, W1_ref[...], (((1,), (0,)), ((), ())),
                        preferred_element_type=jnp.float32)
    h = jnp.maximum(h + b1_ref[...], 0.0)
    s = lax.dot_general(W2_ref[...], h, (((0,), (1,)), ((), ())),
                        preferred_element_type=jnp.float32)
    out_ref[...] = (s + b2_ref[...])[None]


def _scores(feat, coord, W1, b1, W2, b2):
    grid = N // BR
    out = pl.pallas_call(
        _score_body,
        grid=(grid,),
        in_specs=[pl.BlockSpec((BR, D), lambda i: (i, 0)),
                  pl.BlockSpec((BR, 3), lambda i: (i, 0)),
                  pl.BlockSpec((D + 3, D), lambda i: (0, 0)),
                  pl.BlockSpec((1, D), lambda i: (0, 0)),
                  pl.BlockSpec((D, 1), lambda i: (0, 0)),
                  pl.BlockSpec((1, 1), lambda i: (0, 0))],
        out_specs=pl.BlockSpec((1, 1, BR), lambda i: (i, 0, 0)),
        out_shape=jax.ShapeDtypeStruct((grid, 1, BR), jnp.float32),
    )(feat, coord, W1, b1.reshape(1, D), W2, b2.reshape(1, 1))
    return out.reshape(ROWS, 128)


MR = M // 128


def _cmp_first(ka, ia, kb, ib):
    return (ka > kb) | ((ka == kb) & (ia < ib))


def _shift_left_flat(x, s):
    if s < 128:
        a = pltpu.roll(x, 128 - s, 1)
        b = pltpu.roll(a, ROWS - 1, 0)
        c = lax.broadcasted_iota(jnp.int32, x.shape, 1)
        return jnp.where(c < 128 - s, a, b)
    t = s // 128
    return pltpu.roll(x, ROWS - t, 0)


def _partner(x, s):
    if s < 128:
        lo = pltpu.roll(x, 128 - s, 1)
        hi = pltpu.roll(x, s, 1)
        cbit = (lax.broadcasted_iota(jnp.int32, x.shape, 1) & s) == 0
    else:
        t = s // 128
        lo = pltpu.roll(x, x.shape[0] - t, 0)
        hi = pltpu.roll(x, t, 0)
        cbit = (lax.broadcasted_iota(jnp.int32, x.shape, 0) & t) == 0
    return jnp.where(cbit, lo, hi)


def _excl_cumsum_flat(m_f32):
    c1 = lax.broadcasted_iota(jnp.int32, (128, 128), 0)
    c2 = lax.broadcasted_iota(jnp.int32, (128, 128), 1)
    P = (c1 < c2).astype(jnp.float32)
    rowpre = lax.dot_general(m_f32, P, (((1,), (0,)), ((), ())),
                             preferred_element_type=jnp.float32)
    r1 = lax.broadcasted_iota(jnp.int32, (ROWS, ROWS), 0)
    r2 = lax.broadcasted_iota(jnp.int32, (ROWS, ROWS), 1)
    Q = (r2 < r1).astype(jnp.float32)
    rowsum = jnp.sum(m_f32, axis=1, keepdims=True)
    roff = lax.dot_general(Q, rowsum, (((1,), (0,)), ((), ())),
                           preferred_element_type=jnp.float32)
    return rowpre + roff


def _k2_body(score_ref, idx_out_ref):
    sc = score_ref[...]
    b = pltpu.bitcast(sc, jnp.int32)
    u = jnp.where(b < 0, b ^ 0x7FFFFFFF, b)
    ridx = lax.broadcasted_iota(jnp.int32, u.shape, 0)
    cidx = lax.broadcasted_iota(jnp.int32, u.shape, 1)
    idx = ridx * 128 + cidx

    nonneg = jnp.sum((u >= 0).astype(jnp.int32))
    T = jnp.where(nonneg >= M, 0, jnp.int32(-2147483648))
    for bit in range(30, -1, -1):
        cand = T | (1 << bit)
        cnt = jnp.sum((u >= cand).astype(jnp.int32))
        T = jnp.where(cnt >= M, cand, T)

    gt = u > T
    eq = u == T
    extra = (M - jnp.sum(gt.astype(jnp.int32))).astype(jnp.float32)
    eq_rank = _excl_cumsum_flat(eq.astype(jnp.float32))
    win = gt | (eq & (eq_rank < extra))

    wrank = _excl_cumsum_flat(win.astype(jnp.float32)).astype(jnp.int32)
    d = jnp.where(win, idx - wrank, 0)
    winl = win.astype(jnp.int32)
    for bit in range(16):
        s2 = 1 << bit
        iu = _shift_left_flat(u, s2)
        ii = _shift_left_flat(idx, s2)
        iw = _shift_left_flat(winl, s2)
        idd = _shift_left_flat(d, s2)
        move_in = (iw == 1) & (((idd >> bit) & 1) == 1)
        u = jnp.where(move_in, iu, u)
        idx = jnp.where(move_in, ii, idx)
        newd = jnp.where(move_in, idd, d)
        neww = jnp.where(move_in, 1,
                         jnp.where((winl == 1) & (((d >> bit) & 1) == 0),
                                   winl, 0))
        d = newd
        winl = neww

    ku = u[:MR, :]
    ki = idx[:MR, :]
    rr = lax.broadcasted_iota(jnp.int32, ku.shape, 0)
    cc = lax.broadcasted_iota(jnp.int32, ku.shape, 1)
    for kk in range(1, 14):
        k = 1 << kk
        if k < 128:
            region_desc = (cc & k) == 0
        else:
            region_desc = (rr & (k // 128)) == 0
        for j in range(kk - 1, -1, -1):
            st = 1 << j
            if st < 128:
                low = (cc & st) == 0
            else:
                low = (rr & (st // 128)) == 0
            pk = _partner(ku, st)
            pi = _partner(ki, st)
            take_max = region_desc == low
            self_first = _cmp_first(ku, ki, pk, pi)
            keep = take_max == self_first
            ku = jnp.where(keep, ku, pk)
            ki = jnp.where(keep, ki, pi)
    idx_out_ref[...] = ki


def _top_idx(scores):
    return pl.pallas_call(
        _k2_body,
        in_specs=[pl.BlockSpec((ROWS, 128), lambda: (0, 0))],
        out_specs=pl.BlockSpec((MR, 128), lambda: (0, 0)),
        out_shape=jax.ShapeDtypeStruct((MR, 128), jnp.int32),
    )(scores)


@functools.cache
def _make_gather():
    info = plsc.get_sparse_core_info()
    NC, NS = info.num_cores, info.num_subcores
    NW = NC * NS
    b_per_w = M // NW
    CH = 128
    NCH = b_per_w // CH
    mesh = plsc.VectorSubcoreMesh(core_axis_name="c", subcore_axis_name="s")

    @functools.partial(
        pl.kernel, mesh=mesh,
        out_type=[jax.ShapeDtypeStruct((M, D), jnp.float32),
                  jax.ShapeDtypeStruct((M, 128), jnp.float32)],
        scratch_types=[pltpu.VMEM((b_per_w,), jnp.int32),
                       pltpu.VMEM((CH, D), jnp.float32),
                       pltpu.VMEM((b_per_w, 128), jnp.float32),
                       pltpu.SemaphoreType.DMA,
                       pltpu.SemaphoreType.DMA],
    )
    def gather(feat_hbm, coordp_hbm, idx_hbm, out_f_hbm, out_c_hbm,
               idx_v, rows_v, crows_v, sem, csem):
        wid = lax.axis_index("s") * NC + lax.axis_index("c")
        base = wid * b_per_w
        pltpu.sync_copy(idx_hbm.at[pl.ds(base, b_per_w)], idx_v)
        for ch in range(NCH):
            idx_ch = idx_v.at[pl.ds(ch * CH, CH)]
            pltpu.async_copy(feat_hbm.at[idx_ch], rows_v, sem).wait()
            pltpu.sync_copy(rows_v, out_f_hbm.at[pl.ds(base + ch * CH, CH)])
            pltpu.async_copy(coordp_hbm.at[idx_ch],
                             crows_v.at[pl.ds(ch * CH, CH)], csem).wait()
        pltpu.sync_copy(crows_v, out_c_hbm.at[pl.ds(base, b_per_w)])

    return gather


def kernel(feat_list, coord_list, W1, b1, W2, b2):
    feat = feat_list[0]
    coord = coord_list[0]
    scores = _scores(feat, coord, W1, b1, W2, b2)
    top = _top_idx(scores).reshape(M)
    coordp = jnp.pad(coord, ((0, 0), (0, 125)))
    out_f, out_c = _make_gather()(feat, coordp, top)
    return out_f[None], out_c[:, :3][None]
